# baseline (speedup 1.0000x reference)
import functools

import jax
import jax.numpy as jnp
from jax import lax
from jax.experimental import pallas as pl
from jax.experimental.pallas import tpu as pltpu
from jax.experimental.pallas import tpu_sc as plsc

NC = 2
NS = 16
NW = NC * NS
GN = 16
RB = 4


def _sc_gather_sum(atoms2, edges_t, chunk, phase):
  d = atoms2.shape[-1]
  k = edges_t.shape[0]
  npw = chunk // NW
  ng = npw // GN
  assert ng % RB == 0 and npw % 128 == 0
  gl = GN * k

  mesh = plsc.VectorSubcoreMesh(
      core_axis_name="c", subcore_axis_name="s", num_cores=NC, num_subcores=NS)

  @functools.partial(
      pl.kernel,
      out_type=jax.ShapeDtypeStruct((chunk, d), jnp.float32),
      mesh=mesh,
      scratch_types=[
          pltpu.VMEM((k, npw), jnp.int32),
          pltpu.VMEM((ng * gl,), jnp.int32),
          pltpu.VMEM((RB, gl, d), jnp.float32),
          pltpu.VMEM((RB, GN, d), jnp.float32),
          pltpu.SemaphoreType.DMA((RB,)),
          pltpu.SemaphoreType.DMA((RB,)),
      ],
  )
  def sc_kernel(atoms_hbm, edges_hbm, out_hbm,
                edges_v, idx_v, rows_v, out_v, gsem, osem):
    c = lax.axis_index("c")
    s = lax.axis_index("s")
    wid = s * NC + c
    nbase = wid * npw
    gbase = phase * chunk + nbase

    with jax.named_scope("stage_edges"):
      pltpu.sync_copy(edges_hbm.at[:, pl.ds(gbase, npw)], edges_v)

    with jax.named_scope("assemble_idx"):
      def asm(g, carry):
        self16 = (jnp.full((16,), gbase + g * GN, jnp.int32)
                  + lax.iota(jnp.int32, 16))
        for kk in range(k):
          ev = edges_v[kk, pl.ds(g * GN, GN)]
          idx_v[pl.ds(g * gl + kk * GN, GN)] = jnp.where(ev < 0, self16, ev)
        return carry
      lax.fori_loop(0, ng, asm, 0, unroll=2)

    def start_gather(g, slot):
      pltpu.async_copy(atoms_hbm.at[idx_v.at[pl.ds(g * gl, gl)]],
                       rows_v.at[slot], gsem.at[slot])

    def start_store(g, slot):
      pltpu.async_copy(out_v.at[slot],
                       out_hbm.at[pl.ds(nbase + g * GN, GN)], osem.at[slot])

    def wait_gather(slot):
      pltpu.make_async_copy(atoms_hbm.at[idx_v.at[pl.ds(0, gl)]],
                            rows_v.at[slot], gsem.at[slot]).wait()

    def wait_store(slot):
      pltpu.make_async_copy(out_v.at[slot], out_hbm.at[pl.ds(nbase, GN)],
                            osem.at[slot]).wait()

    with jax.named_scope("prime_ring"):
      for r in range(RB):
        start_gather(r, r)

    def compute_group(g, slot):
      def node_body(i, carry):
        row = lambda kk, cc: rows_v[slot, kk * GN + i, pl.ds(cc * 16, 16)]
        for cc in range(d // 16):
          acc = row(0, cc) + row(1, cc)
          acc2 = row(2, cc) + row(3, cc)
          acc3 = row(4, cc) + row(5, cc)
          out_v[slot, i, pl.ds(cc * 16, 16)] = acc + (acc2 + acc3)
        return carry
      lax.fori_loop(0, GN, node_body, 0, unroll=2)

    def outer(o, carry):
      gg = o * RB
      for r in range(RB):
        g = gg + r
        wait_gather(r)

        @pl.when(o > 0)
        def _():
          wait_store(r)

        compute_group(g, r)
        start_store(g, r)

        @pl.when(g + RB < ng)
        def _():
          start_gather(g + RB, r)

      return carry

    with jax.named_scope("mainloop"):
      lax.fori_loop(0, ng // RB, outer, 0)
    with jax.named_scope("drain"):
      for r in range(RB):
        wait_store(r)

  return sc_kernel(atoms2, edges_t)


def _tc_degree_linear(outbuf, sc_sum, atoms2, deg_b, wfull, n_out, tile,
                      chunk, phase):
  d = sc_sum.shape[-1]

  def body(x_ref, a_ref, dg_ref, w_ref, o_ref):
    dg = dg_ref[...]
    coeff = dg.astype(jnp.float32) - 5.0
    x = (x_ref[...] + coeff * a_ref[...]).astype(jnp.bfloat16)
    dgb = jnp.broadcast_to(dg, (tile, d))
    parts = [jnp.where(dgb == float(k), x, jnp.bfloat16(0.0))
             for k in range(6)]
    lane = lax.broadcasted_iota(jnp.int32, (tile, d), 1).astype(jnp.bfloat16)
    parts.append((dgb == lane).astype(jnp.bfloat16))
    xcat = jnp.concatenate(parts, axis=1)
    acc = lax.dot_general(xcat, w_ref[...], (((1,), (0,)), ((), ())),
                          preferred_element_type=jnp.float32)
    o_ref[...] = jnp.maximum(acc, 0.0)

  poff = phase * (chunk // tile)
  nrows = min(chunk, n_out - phase * chunk)
  grid = (nrows + tile - 1) // tile
  in_specs = [
      pl.BlockSpec((tile, d), lambda i: (i, 0)),
      pl.BlockSpec((tile, d), lambda i: (poff + i, 0)),
      pl.BlockSpec((tile, 1), lambda i: (poff + i, 0)),
      pl.BlockSpec((7 * d, d), lambda i: (0, 0)),
  ]
  args = [sc_sum, atoms2, deg_b, wfull]
  aliases = {}
  fn = body
  if outbuf is not None:
    in_specs.insert(0, pl.BlockSpec(memory_space=pltpu.ANY))
    args.insert(0, outbuf)
    aliases = {0: 0}
    fn = lambda _, *refs: body(*refs)
  return pl.pallas_call(
      fn,
      grid=(grid,),
      in_specs=in_specs,
      out_specs=pl.BlockSpec((tile, d), lambda i: (poff + i, 0)),
      out_shape=jax.ShapeDtypeStruct((n_out, d), jnp.float32),
      input_output_aliases=aliases,
  )(*args)


def kernel(atoms, edges, W, b):
  bsz, n, d = atoms.shape
  k = edges.shape[-1]

  align = NW * GN * 8
  n_pad = ((n + align - 1) // align) * align
  tile = 1024
  assert bsz == 1

  atoms2 = atoms[0]
  e = edges[0]
  pad_cols = jnp.broadcast_to(
      (jnp.arange(n, n_pad, dtype=jnp.int32) % n)[None, :], (k, n_pad - n))
  edges_t = jnp.concatenate([e.T, pad_cols], axis=1)

  deg = (e != -1).sum(-1).astype(jnp.bfloat16)
  deg_b = jnp.pad(deg, (0, n_pad - n))[:, None]
  wfull = jnp.concatenate(
      [W.reshape(k * d, d), jnp.pad(b, ((0, d - b.shape[0]), (0, 0)))],
      axis=0).astype(jnp.bfloat16)

  phases = 1
  chunk = n_pad // phases
  assert chunk % tile == 0
  out = None
  for p in range(phases):
    sc_p = _sc_gather_sum(atoms2, edges_t, chunk, p)
    out = _tc_degree_linear(out, sc_p, atoms2, deg_b, wfull, n, tile,
                            chunk, p)
  return out[None]
